# baseline (speedup 1.0000x reference)
import functools

import jax
import jax.numpy as jnp
from jax import lax
from jax.experimental import pallas as pl
from jax.experimental.pallas import tpu as pltpu
from jax.experimental.pallas import tpu_sc as plsc

N = 10000
F = 128
NPAD = 10240
CHUNK = 128
NCH = 80
NT = 32
EPAD = NT * NCH * CHUNK
SHARE = NPAD // 16

_MESH = plsc.VectorSubcoreMesh(core_axis_name="c", subcore_axis_name="s")


@functools.partial(
    pl.kernel,
    out_type=jax.ShapeDtypeStruct((2, NPAD, F), jnp.float32),
    mesh=_MESH,
    scratch_types=[
        pltpu.VMEM((NCH, CHUNK), jnp.int32),
        pltpu.VMEM((CHUNK, F), jnp.float32),
        pltpu.VMEM_SHARED((NPAD, F), jnp.float32),
    ],
)
def _sc_degree(dst_hbm, ones_hbm, zeros_hbm, out_hbm, dst_v, buf_v, acc_sh):
    cid = lax.axis_index("c")
    sid = lax.axis_index("s")
    wid = cid * 16 + sid
    pltpu.sync_copy(dst_hbm.at[wid], dst_v)
    pltpu.sync_copy(zeros_hbm, buf_v)
    for k in range(SHARE // CHUNK):
        pltpu.sync_copy(buf_v, acc_sh.at[pl.ds(sid * SHARE + k * CHUNK, CHUNK)])
    plsc.subcore_barrier()
    pltpu.sync_copy(ones_hbm, buf_v)

    def body(j, carry):
        pltpu.sync_copy(buf_v, acc_sh.at[dst_v.at[j]], add=True)
        return carry

    lax.fori_loop(0, NCH, body, 0)
    plsc.subcore_barrier()
    for k in range(SHARE // CHUNK):
        sl = pl.ds(sid * SHARE + k * CHUNK, CHUNK)
        pltpu.sync_copy(acc_sh.at[sl], buf_v)
        pltpu.sync_copy(buf_v, out_hbm.at[cid, sl])


GCH = 128
T0 = 40
T1 = 120
GNCH = max(T0, T1)
NBUF = 2


@functools.partial(
    pl.kernel,
    out_type=jax.ShapeDtypeStruct((2, NPAD, F), jnp.float32),
    mesh=_MESH,
    scratch_types=[
        pltpu.VMEM((GNCH, GCH), jnp.int32),
        pltpu.VMEM((NBUF, GCH), jnp.int32),
        [pltpu.VMEM((GCH, F), jnp.float32) for _ in range(NBUF)],
        [pltpu.SemaphoreType.DMA for _ in range(NBUF)],
        [pltpu.SemaphoreType.DMA for _ in range(NBUF)],
        pltpu.VMEM_SHARED((NPAD, F), jnp.float32),
    ],
)
def _sc_propagate(y_hbm, src_hbm, dst_hbm, zeros_hbm, out_hbm,
                  src_v, dring, rows, sg, sd, acc_sh):
    cid = lax.axis_index("c")
    sid = lax.axis_index("s")
    wid = cid * 16 + sid
    tc = jnp.where(cid == 0, T0, T1)
    pltpu.sync_copy(src_hbm.at[wid], src_v)
    pltpu.sync_copy(zeros_hbm.at[pl.ds(0, GCH)], rows[0])
    pltpu.sync_copy(zeros_hbm.at[pl.ds(0, GCH)], rows[1])
    for k in range(SHARE // GCH):
        pltpu.sync_copy(rows[k % 2], acc_sh.at[pl.ds(sid * SHARE + k * GCH, GCH)])
    plsc.subcore_barrier()

    for b in range(NBUF):
        pltpu.async_copy(dst_hbm.at[wid, b], dring.at[b], sd[b])
        pltpu.async_copy(y_hbm.at[src_v.at[b]], rows[b], sg[b])

    def body(p, carry):
        for b in range(NBUF):
            j = NBUF * p + b

            @pl.when(j < tc)
            def _():
                nxt = jnp.minimum(j + NBUF, tc - 1)
                pltpu.make_async_copy(y_hbm.at[src_v.at[j]], rows[b], sg[b]).wait()
                pltpu.make_async_copy(dst_hbm.at[wid, j], dring.at[b], sd[b]).wait()
                pltpu.sync_copy(rows[b], acc_sh.at[dring.at[b]], add=True)
                pltpu.async_copy(dst_hbm.at[wid, nxt], dring.at[b], sd[b])
                pltpu.async_copy(y_hbm.at[src_v.at[nxt]], rows[b], sg[b])
        return carry

    lax.fori_loop(0, GNCH // NBUF, body, 0)
    for b in range(NBUF):
        pltpu.make_async_copy(y_hbm.at[src_v.at[tc - 1]], rows[b], sg[b]).wait()
        pltpu.make_async_copy(dst_hbm.at[wid, tc - 1], dring.at[b], sd[b]).wait()
    plsc.subcore_barrier()
    for k in range(SHARE // GCH):
        sl = pl.ds(sid * SHARE + k * GCH, GCH)
        pltpu.sync_copy(acc_sh.at[sl], rows[k % 2])
        pltpu.sync_copy(rows[k % 2], out_hbm.at[cid, sl])


BN = 640
BNO = 400


def _norm_from(deg_ref):
    d = deg_ref[0, :, 0:1] + deg_ref[1, :, 0:1]
    return lax.rsqrt(jnp.maximum(d, 1.0))


def _tc_y1_body(deg_ref, feat_ref, y_ref):
    y_ref[...] = feat_ref[...] * _norm_from(deg_ref)


_tc_y1 = pl.pallas_call(
    _tc_y1_body,
    grid=(NPAD // BN,),
    in_specs=[
        pl.BlockSpec((2, BN, F), lambda i: (0, i, 0)),
        pl.BlockSpec((BN, F), lambda i: (i, 0)),
    ],
    out_specs=pl.BlockSpec((BN, F), lambda i: (i, 0)),
    out_shape=jax.ShapeDtypeStruct((NPAD, F), jnp.float32),
)


def _tc_tx1_body(r_ref, deg_ref, feat_ref, hp_ref, tx1_ref, y2_ref):
    r = r_ref[0, 0]
    nrm = _norm_from(deg_ref)
    h = (hp_ref[0] + hp_ref[1]) * nrm
    tx1 = (r - 1.0) * feat_ref[...] - r * h
    tx1_ref[...] = tx1
    y2_ref[...] = tx1 * nrm


_tc_tx1 = pl.pallas_call(
    _tc_tx1_body,
    grid=(NPAD // BN,),
    in_specs=[
        pl.BlockSpec(memory_space=pltpu.SMEM),
        pl.BlockSpec((2, BN, F), lambda i: (0, i, 0)),
        pl.BlockSpec((BN, F), lambda i: (i, 0)),
        pl.BlockSpec((2, BN, F), lambda i: (0, i, 0)),
    ],
    out_specs=[
        pl.BlockSpec((BN, F), lambda i: (i, 0)),
        pl.BlockSpec((BN, F), lambda i: (i, 0)),
    ],
    out_shape=[
        jax.ShapeDtypeStruct((NPAD, F), jnp.float32),
        jax.ShapeDtypeStruct((NPAD, F), jnp.float32),
    ],
)


def _tc_out_body(r_ref, deg_ref, feat_ref, tx1_ref, hp_ref, w_ref, out_ref):
    r = r_ref[0, 0]
    nrm = _norm_from(deg_ref)
    h2 = (hp_ref[0] + hp_ref[1]) * nrm
    f = feat_ref[...]
    t1 = tx1_ref[...]
    t2 = -2.0 * r * h2 + 2.0 * (r - 1.0) * t1 - f
    x = jnp.concatenate([f, t1, t2], axis=1)
    out_ref[...] = jnp.dot(x, w_ref[...], preferred_element_type=jnp.float32)


_tc_out = pl.pallas_call(
    _tc_out_body,
    grid=(N // BNO,),
    in_specs=[
        pl.BlockSpec(memory_space=pltpu.SMEM),
        pl.BlockSpec((2, BNO, F), lambda i: (0, i, 0)),
        pl.BlockSpec((BNO, F), lambda i: (i, 0)),
        pl.BlockSpec((BNO, F), lambda i: (i, 0)),
        pl.BlockSpec((2, BNO, F), lambda i: (0, i, 0)),
        pl.BlockSpec((3 * F, F), lambda i: (0, 0)),
    ],
    out_specs=pl.BlockSpec((BNO, F), lambda i: (i, 0)),
    out_shape=jax.ShapeDtypeStruct((N, F), jnp.float32),
)


def kernel(feat, edge_index, lambda_max, W0, W1, W2):
    src = edge_index[0].astype(jnp.int32)
    dst = edge_index[1].astype(jnp.int32)
    e = src.shape[0]
    pad = jnp.full((EPAD - e,), N, jnp.int32)

    def shard(flat):
        cut = 16 * T0 * GCH
        p0 = flat[:cut].reshape(16, T0, GCH)
        p0 = jnp.pad(p0, ((0, 0), (0, GNCH - T0), (0, 0)), constant_values=N)
        p1 = flat[cut:].reshape(16, T1, GCH)
        p1 = jnp.pad(p1, ((0, 0), (0, GNCH - T1), (0, 0)), constant_values=N)
        return jnp.concatenate([p0, p1], axis=0)

    src_flat = jnp.concatenate([src, pad])
    dst_flat = jnp.concatenate([dst, pad])
    src_t = shard(src_flat)
    dst_t = shard(dst_flat)
    dst_deg = dst_flat.reshape(NT, NCH, CHUNK)
    feat_pad = jnp.pad(feat, ((0, NPAD - N), (0, 0)))
    ones128 = jnp.ones((CHUNK, F), jnp.float32)
    zeros128 = jnp.zeros((CHUNK, F), jnp.float32)
    r = jnp.reshape((2.0 / lambda_max).astype(jnp.float32), (1, 1))

    deg2 = _sc_degree(dst_deg, ones128, zeros128)
    y1 = _tc_y1(deg2, feat_pad)
    h1 = _sc_propagate(y1, src_t, dst_t, zeros128)
    tx1, y2 = _tc_tx1(r, deg2, feat_pad, h1)
    h2 = _sc_propagate(y2, src_t, dst_t, zeros128)
    wcat = jnp.concatenate([W0, W1, W2], axis=0)
    return _tc_out(r, deg2, feat, tx1, h2, wcat)


# speedup vs baseline: 1.0904x; 1.0904x over previous
import functools

import jax
import jax.numpy as jnp
from jax import lax
from jax.experimental import pallas as pl
from jax.experimental.pallas import tpu as pltpu
from jax.experimental.pallas import tpu_sc as plsc

N = 10000
F = 128
NPAD = 10240
CHUNK = 128
NCH = 80
NT = 32
EPAD = NT * NCH * CHUNK
SHARE = NPAD // 16

_MESH = plsc.VectorSubcoreMesh(core_axis_name="c", subcore_axis_name="s")


@functools.partial(
    pl.kernel,
    out_type=jax.ShapeDtypeStruct((2, NPAD, F), jnp.float32),
    mesh=_MESH,
    scratch_types=[
        pltpu.VMEM((NCH, CHUNK), jnp.int32),
        pltpu.VMEM((CHUNK, F), jnp.float32),
        pltpu.VMEM_SHARED((NPAD, F), jnp.float32),
    ],
)
def _sc_degree(dst_hbm, ones_hbm, zeros_hbm, out_hbm, dst_v, buf_v, acc_sh):
    cid = lax.axis_index("c")
    sid = lax.axis_index("s")
    wid = cid * 16 + sid
    pltpu.sync_copy(dst_hbm.at[wid], dst_v)
    pltpu.sync_copy(zeros_hbm, buf_v)
    for k in range(SHARE // CHUNK):
        pltpu.sync_copy(buf_v, acc_sh.at[pl.ds(sid * SHARE + k * CHUNK, CHUNK)])
    plsc.subcore_barrier()
    pltpu.sync_copy(ones_hbm, buf_v)

    def body(j, carry):
        pltpu.sync_copy(buf_v, acc_sh.at[dst_v.at[j]], add=True)
        return carry

    lax.fori_loop(0, NCH, body, 0)
    plsc.subcore_barrier()
    for k in range(SHARE // CHUNK):
        sl = pl.ds(sid * SHARE + k * CHUNK, CHUNK)
        pltpu.sync_copy(acc_sh.at[sl], buf_v)
        pltpu.sync_copy(buf_v, out_hbm.at[cid, sl])


GCH = 128
T0 = 120
T1 = 40
GNCH = max(T0, T1)
NBUF = 2


@functools.partial(
    pl.kernel,
    out_type=jax.ShapeDtypeStruct((2, NPAD, F), jnp.float32),
    mesh=_MESH,
    scratch_types=[
        pltpu.VMEM((GNCH, GCH), jnp.int32),
        pltpu.VMEM((NBUF, GCH), jnp.int32),
        [pltpu.VMEM((GCH, F), jnp.float32) for _ in range(NBUF)],
        [pltpu.SemaphoreType.DMA for _ in range(NBUF)],
        [pltpu.SemaphoreType.DMA for _ in range(NBUF)],
        pltpu.VMEM_SHARED((NPAD, F), jnp.float32),
    ],
)
def _sc_propagate(y_hbm, src_hbm, dst_hbm, zeros_hbm, out_hbm,
                  src_v, dring, rows, sg, sd, acc_sh):
    cid = lax.axis_index("c")
    sid = lax.axis_index("s")
    wid = cid * 16 + sid
    tc = jnp.where(cid == 0, T0, T1)
    pltpu.sync_copy(src_hbm.at[wid], src_v)
    pltpu.sync_copy(zeros_hbm.at[pl.ds(0, GCH)], rows[0])
    pltpu.sync_copy(zeros_hbm.at[pl.ds(0, GCH)], rows[1])
    for k in range(SHARE // GCH):
        pltpu.sync_copy(rows[k % 2], acc_sh.at[pl.ds(sid * SHARE + k * GCH, GCH)])
    plsc.subcore_barrier()

    for b in range(NBUF):
        pltpu.async_copy(dst_hbm.at[wid, b], dring.at[b], sd[b])
        pltpu.async_copy(y_hbm.at[src_v.at[b]], rows[b], sg[b])

    def body(p, carry):
        for b in range(NBUF):
            j = NBUF * p + b

            @pl.when(j < tc)
            def _():
                nxt = jnp.minimum(j + NBUF, tc - 1)
                pltpu.make_async_copy(y_hbm.at[src_v.at[j]], rows[b], sg[b]).wait()
                pltpu.make_async_copy(dst_hbm.at[wid, j], dring.at[b], sd[b]).wait()
                pltpu.sync_copy(rows[b], acc_sh.at[dring.at[b]], add=True)
                pltpu.async_copy(dst_hbm.at[wid, nxt], dring.at[b], sd[b])
                pltpu.async_copy(y_hbm.at[src_v.at[nxt]], rows[b], sg[b])
        return carry

    lax.fori_loop(0, GNCH // NBUF, body, 0)
    for b in range(NBUF):
        pltpu.make_async_copy(y_hbm.at[src_v.at[tc - 1]], rows[b], sg[b]).wait()
        pltpu.make_async_copy(dst_hbm.at[wid, tc - 1], dring.at[b], sd[b]).wait()
    plsc.subcore_barrier()
    for k in range(SHARE // GCH):
        sl = pl.ds(sid * SHARE + k * GCH, GCH)
        pltpu.sync_copy(acc_sh.at[sl], rows[k % 2])
        pltpu.sync_copy(rows[k % 2], out_hbm.at[cid, sl])


BN = 640
BNO = 400


def _norm_from(deg_ref):
    d = deg_ref[0, :, 0:1] + deg_ref[1, :, 0:1]
    return lax.rsqrt(jnp.maximum(d, 1.0))


def _tc_y1_body(deg_ref, feat_ref, y_ref):
    y_ref[...] = feat_ref[...] * _norm_from(deg_ref)


_tc_y1 = pl.pallas_call(
    _tc_y1_body,
    grid=(NPAD // BN,),
    in_specs=[
        pl.BlockSpec((2, BN, F), lambda i: (0, i, 0)),
        pl.BlockSpec((BN, F), lambda i: (i, 0)),
    ],
    out_specs=pl.BlockSpec((BN, F), lambda i: (i, 0)),
    out_shape=jax.ShapeDtypeStruct((NPAD, F), jnp.float32),
)


def _tc_tx1_body(r_ref, deg_ref, feat_ref, hp_ref, tx1_ref, y2_ref):
    r = r_ref[0, 0]
    nrm = _norm_from(deg_ref)
    h = (hp_ref[0] + hp_ref[1]) * nrm
    tx1 = (r - 1.0) * feat_ref[...] - r * h
    tx1_ref[...] = tx1
    y2_ref[...] = tx1 * nrm


_tc_tx1 = pl.pallas_call(
    _tc_tx1_body,
    grid=(NPAD // BN,),
    in_specs=[
        pl.BlockSpec(memory_space=pltpu.SMEM),
        pl.BlockSpec((2, BN, F), lambda i: (0, i, 0)),
        pl.BlockSpec((BN, F), lambda i: (i, 0)),
        pl.BlockSpec((2, BN, F), lambda i: (0, i, 0)),
    ],
    out_specs=[
        pl.BlockSpec((BN, F), lambda i: (i, 0)),
        pl.BlockSpec((BN, F), lambda i: (i, 0)),
    ],
    out_shape=[
        jax.ShapeDtypeStruct((NPAD, F), jnp.float32),
        jax.ShapeDtypeStruct((NPAD, F), jnp.float32),
    ],
)


def _tc_out_body(r_ref, deg_ref, feat_ref, tx1_ref, hp_ref, w_ref, out_ref):
    r = r_ref[0, 0]
    nrm = _norm_from(deg_ref)
    h2 = (hp_ref[0] + hp_ref[1]) * nrm
    f = feat_ref[...]
    t1 = tx1_ref[...]
    t2 = -2.0 * r * h2 + 2.0 * (r - 1.0) * t1 - f
    x = jnp.concatenate([f, t1, t2], axis=1)
    out_ref[...] = jnp.dot(x, w_ref[...], preferred_element_type=jnp.float32)


_tc_out = pl.pallas_call(
    _tc_out_body,
    grid=(N // BNO,),
    in_specs=[
        pl.BlockSpec(memory_space=pltpu.SMEM),
        pl.BlockSpec((2, BNO, F), lambda i: (0, i, 0)),
        pl.BlockSpec((BNO, F), lambda i: (i, 0)),
        pl.BlockSpec((BNO, F), lambda i: (i, 0)),
        pl.BlockSpec((2, BNO, F), lambda i: (0, i, 0)),
        pl.BlockSpec((3 * F, F), lambda i: (0, 0)),
    ],
    out_specs=pl.BlockSpec((BNO, F), lambda i: (i, 0)),
    out_shape=jax.ShapeDtypeStruct((N, F), jnp.float32),
)


def kernel(feat, edge_index, lambda_max, W0, W1, W2):
    src = edge_index[0].astype(jnp.int32)
    dst = edge_index[1].astype(jnp.int32)
    e = src.shape[0]
    pad = jnp.full((EPAD - e,), N, jnp.int32)

    def shard(flat):
        cut = 16 * T0 * GCH
        p0 = flat[:cut].reshape(16, T0, GCH)
        p0 = jnp.pad(p0, ((0, 0), (0, GNCH - T0), (0, 0)), constant_values=N)
        p1 = flat[cut:].reshape(16, T1, GCH)
        p1 = jnp.pad(p1, ((0, 0), (0, GNCH - T1), (0, 0)), constant_values=N)
        return jnp.concatenate([p0, p1], axis=0)

    src_flat = jnp.concatenate([src, pad])
    dst_flat = jnp.concatenate([dst, pad])
    src_t = shard(src_flat)
    dst_t = shard(dst_flat)
    dst_deg = dst_flat.reshape(NT, NCH, CHUNK)
    feat_pad = jnp.pad(feat, ((0, NPAD - N), (0, 0)))
    ones128 = jnp.ones((CHUNK, F), jnp.float32)
    zeros128 = jnp.zeros((CHUNK, F), jnp.float32)
    r = jnp.reshape((2.0 / lambda_max).astype(jnp.float32), (1, 1))

    deg2 = _sc_degree(dst_deg, ones128, zeros128)
    y1 = _tc_y1(deg2, feat_pad)
    h1 = _sc_propagate(y1, src_t, dst_t, zeros128)
    tx1, y2 = _tc_tx1(r, deg2, feat_pad, h1)
    h2 = _sc_propagate(y2, src_t, dst_t, zeros128)
    wcat = jnp.concatenate([W0, W1, W2], axis=0)
    return _tc_out(r, deg2, feat, tx1, h2, wcat)


# speedup vs baseline: 1.0914x; 1.0010x over previous
import functools

import jax
import jax.numpy as jnp
from jax import lax
from jax.experimental import pallas as pl
from jax.experimental.pallas import tpu as pltpu
from jax.experimental.pallas import tpu_sc as plsc

N = 10000
F = 128
NPAD = 10240
CHUNK = 128
NCH = 80
NT = 32
EPAD = NT * NCH * CHUNK
SHARE = NPAD // 16

_MESH = plsc.VectorSubcoreMesh(core_axis_name="c", subcore_axis_name="s")


@functools.partial(
    pl.kernel,
    out_type=jax.ShapeDtypeStruct((2, NPAD, F), jnp.float32),
    mesh=_MESH,
    scratch_types=[
        pltpu.VMEM((NCH, CHUNK), jnp.int32),
        pltpu.VMEM((CHUNK, F), jnp.float32),
        pltpu.VMEM_SHARED((NPAD, F), jnp.float32),
    ],
)
def _sc_degree(dst_hbm, ones_hbm, zeros_hbm, out_hbm, dst_v, buf_v, acc_sh):
    cid = lax.axis_index("c")
    sid = lax.axis_index("s")
    wid = cid * 16 + sid
    pltpu.sync_copy(dst_hbm.at[wid], dst_v)
    pltpu.sync_copy(zeros_hbm, buf_v)
    for k in range(SHARE // CHUNK):
        pltpu.sync_copy(buf_v, acc_sh.at[pl.ds(sid * SHARE + k * CHUNK, CHUNK)])
    plsc.subcore_barrier()
    pltpu.sync_copy(ones_hbm, buf_v)

    def body(j, carry):
        pltpu.sync_copy(buf_v, acc_sh.at[dst_v.at[j]], add=True)
        return carry

    lax.fori_loop(0, NCH, body, 0)
    plsc.subcore_barrier()
    for k in range(SHARE // CHUNK):
        sl = pl.ds(sid * SHARE + k * CHUNK, CHUNK)
        pltpu.sync_copy(acc_sh.at[sl], buf_v)
        pltpu.sync_copy(buf_v, out_hbm.at[cid, sl])


GCH = 128
T0 = 120
T1 = 40
GNCH = max(T0, T1)
NBUF = 2


@functools.partial(
    pl.kernel,
    out_type=jax.ShapeDtypeStruct((2, NPAD, F), jnp.float32),
    mesh=_MESH,
    scratch_types=[
        pltpu.VMEM((GNCH, GCH), jnp.int32),
        pltpu.VMEM((NBUF, GCH), jnp.int32),
        [pltpu.VMEM((GCH, F), jnp.float32) for _ in range(NBUF)],
        [pltpu.SemaphoreType.DMA for _ in range(NBUF)],
        [pltpu.SemaphoreType.DMA for _ in range(NBUF)],
        pltpu.VMEM_SHARED((NPAD, F), jnp.float32),
    ],
)
def _sc_propagate(y_hbm, src_hbm, dst_hbm, zeros_hbm, out_hbm,
                  src_v, dring, rows, sg, sd, acc_sh):
    cid = lax.axis_index("c")
    sid = lax.axis_index("s")
    wid = cid * 16 + sid
    pltpu.sync_copy(src_hbm.at[wid], src_v)
    pltpu.sync_copy(zeros_hbm.at[pl.ds(0, GCH)], rows[0])
    pltpu.sync_copy(zeros_hbm.at[pl.ds(0, GCH)], rows[1])
    for k in range(SHARE // GCH):
        pltpu.sync_copy(rows[k % 2], acc_sh.at[pl.ds(sid * SHARE + k * GCH, GCH)])
    plsc.subcore_barrier()

    def pipe(lo, hi):
        for b in range(NBUF):
            pltpu.async_copy(dst_hbm.at[wid, lo + b], dring.at[b], sd[b])
            pltpu.async_copy(y_hbm.at[src_v.at[lo + b]], rows[b], sg[b])

        def body(p, carry):
            for b in range(NBUF):
                j = lo + NBUF * p + b
                nxt = jnp.minimum(j + NBUF, hi - 1)
                pltpu.make_async_copy(y_hbm.at[src_v.at[j]], rows[b], sg[b]).wait()
                pltpu.make_async_copy(dst_hbm.at[wid, j], dring.at[b], sd[b]).wait()
                pltpu.sync_copy(rows[b], acc_sh.at[dring.at[b]], add=True)
                pltpu.async_copy(dst_hbm.at[wid, nxt], dring.at[b], sd[b])
                pltpu.async_copy(y_hbm.at[src_v.at[nxt]], rows[b], sg[b])
            return carry

        lax.fori_loop(0, (hi - lo) // NBUF, body, 0)
        for b in range(NBUF):
            pltpu.make_async_copy(y_hbm.at[src_v.at[hi - 1]], rows[b], sg[b]).wait()
            pltpu.make_async_copy(dst_hbm.at[wid, hi - 1], dring.at[b], sd[b]).wait()

    pipe(0, T1)

    @pl.when(cid == 0)
    def _():
        pipe(T1, T0)

    plsc.subcore_barrier()
    for k in range(SHARE // GCH):
        sl = pl.ds(sid * SHARE + k * GCH, GCH)
        pltpu.sync_copy(acc_sh.at[sl], rows[k % 2])
        pltpu.sync_copy(rows[k % 2], out_hbm.at[cid, sl])


BN = 640
BNO = 400


def _norm_from(deg_ref):
    d = deg_ref[0, :, 0:1] + deg_ref[1, :, 0:1]
    return lax.rsqrt(jnp.maximum(d, 1.0))


def _tc_y1_body(deg_ref, feat_ref, y_ref):
    y_ref[...] = feat_ref[...] * _norm_from(deg_ref)


_tc_y1 = pl.pallas_call(
    _tc_y1_body,
    grid=(NPAD // BN,),
    in_specs=[
        pl.BlockSpec((2, BN, F), lambda i: (0, i, 0)),
        pl.BlockSpec((BN, F), lambda i: (i, 0)),
    ],
    out_specs=pl.BlockSpec((BN, F), lambda i: (i, 0)),
    out_shape=jax.ShapeDtypeStruct((NPAD, F), jnp.float32),
)


def _tc_tx1_body(r_ref, deg_ref, feat_ref, hp_ref, tx1_ref, y2_ref):
    r = r_ref[0, 0]
    nrm = _norm_from(deg_ref)
    h = (hp_ref[0] + hp_ref[1]) * nrm
    tx1 = (r - 1.0) * feat_ref[...] - r * h
    tx1_ref[...] = tx1
    y2_ref[...] = tx1 * nrm


_tc_tx1 = pl.pallas_call(
    _tc_tx1_body,
    grid=(NPAD // BN,),
    in_specs=[
        pl.BlockSpec(memory_space=pltpu.SMEM),
        pl.BlockSpec((2, BN, F), lambda i: (0, i, 0)),
        pl.BlockSpec((BN, F), lambda i: (i, 0)),
        pl.BlockSpec((2, BN, F), lambda i: (0, i, 0)),
    ],
    out_specs=[
        pl.BlockSpec((BN, F), lambda i: (i, 0)),
        pl.BlockSpec((BN, F), lambda i: (i, 0)),
    ],
    out_shape=[
        jax.ShapeDtypeStruct((NPAD, F), jnp.float32),
        jax.ShapeDtypeStruct((NPAD, F), jnp.float32),
    ],
)


def _tc_out_body(r_ref, deg_ref, feat_ref, tx1_ref, hp_ref, w_ref, out_ref):
    r = r_ref[0, 0]
    nrm = _norm_from(deg_ref)
    h2 = (hp_ref[0] + hp_ref[1]) * nrm
    f = feat_ref[...]
    t1 = tx1_ref[...]
    t2 = -2.0 * r * h2 + 2.0 * (r - 1.0) * t1 - f
    x = jnp.concatenate([f, t1, t2], axis=1)
    out_ref[...] = jnp.dot(x, w_ref[...], preferred_element_type=jnp.float32)


_tc_out = pl.pallas_call(
    _tc_out_body,
    grid=(N // BNO,),
    in_specs=[
        pl.BlockSpec(memory_space=pltpu.SMEM),
        pl.BlockSpec((2, BNO, F), lambda i: (0, i, 0)),
        pl.BlockSpec((BNO, F), lambda i: (i, 0)),
        pl.BlockSpec((BNO, F), lambda i: (i, 0)),
        pl.BlockSpec((2, BNO, F), lambda i: (0, i, 0)),
        pl.BlockSpec((3 * F, F), lambda i: (0, 0)),
    ],
    out_specs=pl.BlockSpec((BNO, F), lambda i: (i, 0)),
    out_shape=jax.ShapeDtypeStruct((N, F), jnp.float32),
)


def kernel(feat, edge_index, lambda_max, W0, W1, W2):
    src = edge_index[0].astype(jnp.int32)
    dst = edge_index[1].astype(jnp.int32)
    e = src.shape[0]
    pad = jnp.full((EPAD - e,), N, jnp.int32)

    def shard(flat):
        cut = 16 * T0 * GCH
        p0 = flat[:cut].reshape(16, T0, GCH)
        p0 = jnp.pad(p0, ((0, 0), (0, GNCH - T0), (0, 0)), constant_values=N)
        p1 = flat[cut:].reshape(16, T1, GCH)
        p1 = jnp.pad(p1, ((0, 0), (0, GNCH - T1), (0, 0)), constant_values=N)
        return jnp.concatenate([p0, p1], axis=0)

    src_flat = jnp.concatenate([src, pad])
    dst_flat = jnp.concatenate([dst, pad])
    src_t = shard(src_flat)
    dst_t = shard(dst_flat)
    dst_deg = dst_flat.reshape(NT, NCH, CHUNK)
    feat_pad = jnp.pad(feat, ((0, NPAD - N), (0, 0)))
    ones128 = jnp.ones((CHUNK, F), jnp.float32)
    zeros128 = jnp.zeros((CHUNK, F), jnp.float32)
    r = jnp.reshape((2.0 / lambda_max).astype(jnp.float32), (1, 1))

    deg2 = _sc_degree(dst_deg, ones128, zeros128)
    y1 = _tc_y1(deg2, feat_pad)
    h1 = _sc_propagate(y1, src_t, dst_t, zeros128)
    tx1, y2 = _tc_tx1(r, deg2, feat_pad, h1)
    h2 = _sc_propagate(y2, src_t, dst_t, zeros128)
    wcat = jnp.concatenate([W0, W1, W2], axis=0)
    return _tc_out(r, deg2, feat, tx1, h2, wcat)


# speedup vs baseline: 1.4333x; 1.3133x over previous
import functools

import jax
import jax.numpy as jnp
from jax import lax
from jax.experimental import pallas as pl
from jax.experimental.pallas import tpu as pltpu
from jax.experimental.pallas import tpu_sc as plsc

N = 10000
F = 128
NPAD = 10240
CHUNK = 128
NCH = 80
NT = 32
EPAD = NT * NCH * CHUNK
SHARE = NPAD // 16

_MESH = plsc.VectorSubcoreMesh(core_axis_name="c", subcore_axis_name="s")


@functools.partial(
    pl.kernel,
    out_type=jax.ShapeDtypeStruct((2, NPAD, F), jnp.float32),
    mesh=_MESH,
    scratch_types=[
        pltpu.VMEM((NCH, CHUNK), jnp.int32),
        pltpu.VMEM((CHUNK, F), jnp.float32),
        pltpu.VMEM_SHARED((NPAD, F), jnp.float32),
    ],
)
def _sc_degree(dst_hbm, ones_hbm, zeros_hbm, out_hbm, dst_v, buf_v, acc_sh):
    cid = lax.axis_index("c")
    sid = lax.axis_index("s")
    wid = cid * 16 + sid
    pltpu.sync_copy(dst_hbm.at[wid], dst_v)
    pltpu.sync_copy(zeros_hbm, buf_v)
    for k in range(SHARE // CHUNK):
        pltpu.sync_copy(buf_v, acc_sh.at[pl.ds(sid * SHARE + k * CHUNK, CHUNK)])
    plsc.subcore_barrier()
    pltpu.sync_copy(ones_hbm, buf_v)

    def body(j, carry):
        pltpu.sync_copy(buf_v, acc_sh.at[dst_v.at[j]], add=True)
        return carry

    lax.fori_loop(0, NCH, body, 0)
    plsc.subcore_barrier()
    for k in range(SHARE // CHUNK):
        sl = pl.ds(sid * SHARE + k * CHUNK, CHUNK)
        pltpu.sync_copy(acc_sh.at[sl], buf_v)
        pltpu.sync_copy(buf_v, out_hbm.at[cid, sl])


GCH = 128
T0 = 80
T1 = 80
GNCH = max(T0, T1)
NBUF = 2


@functools.partial(
    pl.kernel,
    out_type=jax.ShapeDtypeStruct((2, NPAD, F), jnp.float32),
    mesh=_MESH,
    scratch_types=[
        pltpu.VMEM((GNCH, GCH), jnp.int32),
        pltpu.VMEM((NBUF, GCH), jnp.int32),
        [pltpu.VMEM((GCH, F), jnp.float32) for _ in range(NBUF)],
        [pltpu.SemaphoreType.DMA for _ in range(NBUF)],
        [pltpu.SemaphoreType.DMA for _ in range(NBUF)],
        pltpu.VMEM_SHARED((NPAD, F), jnp.float32),
    ],
)
def _sc_propagate(y_hbm, src_hbm, dst_hbm, zeros_hbm, out_hbm,
                  src_v, dring, rows, sg, sd, acc_sh):
    cid = lax.axis_index("c")
    sid = lax.axis_index("s")
    wid = cid * 16 + sid
    pltpu.sync_copy(src_hbm.at[wid], src_v)
    pltpu.sync_copy(zeros_hbm.at[pl.ds(0, GCH)], rows[0])
    pltpu.sync_copy(zeros_hbm.at[pl.ds(0, GCH)], rows[1])
    for k in range(SHARE // GCH):
        pltpu.sync_copy(rows[k % 2], acc_sh.at[pl.ds(sid * SHARE + k * GCH, GCH)])
    plsc.subcore_barrier()

    def pipe(lo, hi):
        for b in range(NBUF):
            pltpu.async_copy(dst_hbm.at[wid, lo + b], dring.at[b], sd[b])
            pltpu.async_copy(y_hbm.at[src_v.at[lo + b]], rows[b], sg[b])

        def body(p, carry):
            for b in range(NBUF):
                j = lo + NBUF * p + b
                nxt = jnp.minimum(j + NBUF, hi - 1)
                pltpu.make_async_copy(y_hbm.at[src_v.at[j]], rows[b], sg[b]).wait()
                pltpu.make_async_copy(dst_hbm.at[wid, j], dring.at[b], sd[b]).wait()
                pltpu.sync_copy(rows[b], acc_sh.at[dring.at[b]], add=True)
                pltpu.async_copy(dst_hbm.at[wid, nxt], dring.at[b], sd[b])
                pltpu.async_copy(y_hbm.at[src_v.at[nxt]], rows[b], sg[b])
            return carry

        lax.fori_loop(0, (hi - lo) // NBUF, body, 0)
        for b in range(NBUF):
            pltpu.make_async_copy(y_hbm.at[src_v.at[hi - 1]], rows[b], sg[b]).wait()
            pltpu.make_async_copy(dst_hbm.at[wid, hi - 1], dring.at[b], sd[b]).wait()

    pipe(0, T1)

    if T0 > T1:
        @pl.when(cid == 0)
        def _():
            pipe(T1, T0)

    plsc.subcore_barrier()
    for k in range(SHARE // GCH):
        sl = pl.ds(sid * SHARE + k * GCH, GCH)
        pltpu.sync_copy(acc_sh.at[sl], rows[k % 2])
        pltpu.sync_copy(rows[k % 2], out_hbm.at[cid, sl])


BN = 640
BNO = 400


def _norm_from(deg_ref):
    d = deg_ref[0, :, 0:1] + deg_ref[1, :, 0:1]
    return lax.rsqrt(jnp.maximum(d, 1.0))


def _tc_y1_body(deg_ref, feat_ref, y_ref):
    y_ref[...] = feat_ref[...] * _norm_from(deg_ref)


_tc_y1 = pl.pallas_call(
    _tc_y1_body,
    grid=(NPAD // BN,),
    in_specs=[
        pl.BlockSpec((2, BN, F), lambda i: (0, i, 0)),
        pl.BlockSpec((BN, F), lambda i: (i, 0)),
    ],
    out_specs=pl.BlockSpec((BN, F), lambda i: (i, 0)),
    out_shape=jax.ShapeDtypeStruct((NPAD, F), jnp.float32),
)


def _tc_tx1_body(r_ref, deg_ref, feat_ref, hp_ref, tx1_ref, y2_ref):
    r = r_ref[0, 0]
    nrm = _norm_from(deg_ref)
    h = (hp_ref[0] + hp_ref[1]) * nrm
    tx1 = (r - 1.0) * feat_ref[...] - r * h
    tx1_ref[...] = tx1
    y2_ref[...] = tx1 * nrm


_tc_tx1 = pl.pallas_call(
    _tc_tx1_body,
    grid=(NPAD // BN,),
    in_specs=[
        pl.BlockSpec(memory_space=pltpu.SMEM),
        pl.BlockSpec((2, BN, F), lambda i: (0, i, 0)),
        pl.BlockSpec((BN, F), lambda i: (i, 0)),
        pl.BlockSpec((2, BN, F), lambda i: (0, i, 0)),
    ],
    out_specs=[
        pl.BlockSpec((BN, F), lambda i: (i, 0)),
        pl.BlockSpec((BN, F), lambda i: (i, 0)),
    ],
    out_shape=[
        jax.ShapeDtypeStruct((NPAD, F), jnp.float32),
        jax.ShapeDtypeStruct((NPAD, F), jnp.float32),
    ],
)


def _tc_out_body(r_ref, deg_ref, feat_ref, tx1_ref, hp_ref, w_ref, out_ref):
    r = r_ref[0, 0]
    nrm = _norm_from(deg_ref)
    h2 = (hp_ref[0] + hp_ref[1]) * nrm
    f = feat_ref[...]
    t1 = tx1_ref[...]
    t2 = -2.0 * r * h2 + 2.0 * (r - 1.0) * t1 - f
    x = jnp.concatenate([f, t1, t2], axis=1)
    out_ref[...] = jnp.dot(x, w_ref[...], preferred_element_type=jnp.float32)


_tc_out = pl.pallas_call(
    _tc_out_body,
    grid=(N // BNO,),
    in_specs=[
        pl.BlockSpec(memory_space=pltpu.SMEM),
        pl.BlockSpec((2, BNO, F), lambda i: (0, i, 0)),
        pl.BlockSpec((BNO, F), lambda i: (i, 0)),
        pl.BlockSpec((BNO, F), lambda i: (i, 0)),
        pl.BlockSpec((2, BNO, F), lambda i: (0, i, 0)),
        pl.BlockSpec((3 * F, F), lambda i: (0, 0)),
    ],
    out_specs=pl.BlockSpec((BNO, F), lambda i: (i, 0)),
    out_shape=jax.ShapeDtypeStruct((N, F), jnp.float32),
)


def kernel(feat, edge_index, lambda_max, W0, W1, W2):
    src = edge_index[0].astype(jnp.int32)
    dst = edge_index[1].astype(jnp.int32)
    e = src.shape[0]
    pad = jnp.full((EPAD - e,), N, jnp.int32)

    def shard(flat):
        cut = 16 * T0 * GCH
        p0 = flat[:cut].reshape(16, T0, GCH)
        p0 = jnp.pad(p0, ((0, 0), (0, GNCH - T0), (0, 0)), constant_values=N)
        p1 = flat[cut:].reshape(16, T1, GCH)
        p1 = jnp.pad(p1, ((0, 0), (0, GNCH - T1), (0, 0)), constant_values=N)
        return jnp.concatenate([p0, p1], axis=0)

    src_flat = jnp.concatenate([src, pad])
    dst_flat = jnp.concatenate([dst, pad])
    src_t = shard(src_flat)
    dst_t = shard(dst_flat)
    dst_deg = dst_flat.reshape(NT, NCH, CHUNK)
    feat_pad = jnp.pad(feat, ((0, NPAD - N), (0, 0)))
    ones128 = jnp.ones((CHUNK, F), jnp.float32)
    zeros128 = jnp.zeros((CHUNK, F), jnp.float32)
    r = jnp.reshape((2.0 / lambda_max).astype(jnp.float32), (1, 1))

    deg2 = _sc_degree(dst_deg, ones128, zeros128)
    y1 = _tc_y1(deg2, feat_pad)
    h1 = _sc_propagate(y1, src_t, dst_t, zeros128)
    tx1, y2 = _tc_tx1(r, deg2, feat_pad, h1)
    h2 = _sc_propagate(y2, src_t, dst_t, zeros128)
    wcat = jnp.concatenate([W0, W1, W2], axis=0)
    return _tc_out(r, deg2, feat, tx1, h2, wcat)
